# Initial kernel scaffold; baseline (speedup 1.0000x reference)
import jax
import jax.numpy as jnp
from jax.experimental import pallas as pl


def kernel(x, adj, W_e0, a_src_e0, a_dst_e0, W_d0, a_src_d0, a_dst_d0):
    raise NotImplementedError("write your pallas kernel here")


# speedup vs baseline: 1.0592x; 1.0592x over previous
import functools

import jax
import jax.numpy as jnp
from jax.experimental import pallas as pl
from jax.experimental.pallas import tpu as pltpu


def _proj_body(x_ref, w_ref, a_ref, h_ref, f_ref):
    h = jnp.dot(x_ref[...], w_ref[...], preferred_element_type=jnp.float32)
    h_ref[...] = h
    f_ref[...] = jnp.dot(h, a_ref[...], preferred_element_type=jnp.float32)


def _proj(x, W, a_src, block_rows):
    N, d_in = x.shape
    d_out = W.shape[1]
    grid = (pl.cdiv(N, block_rows),)
    return pl.pallas_call(
        _proj_body,
        grid=grid,
        in_specs=[
            pl.BlockSpec((block_rows, d_in), lambda i: (i, 0)),
            pl.BlockSpec((d_in, d_out), lambda i: (0, 0)),
            pl.BlockSpec((d_out, 1), lambda i: (0, 0)),
        ],
        out_specs=[
            pl.BlockSpec((block_rows, d_out), lambda i: (i, 0)),
            pl.BlockSpec((block_rows, 1), lambda i: (i, 0)),
        ],
        out_shape=[
            jax.ShapeDtypeStruct((N, d_out), jnp.float32),
            jax.ShapeDtypeStruct((N, 1), jnp.float32),
        ],
    )(x, W, a_src.reshape(d_out, 1))


def _flash_body(n, bc, n_col_blocks, emit_mask,
                f_src_ref, h_ref, a_dst_ref, adj_ref,
                *refs):
    if emit_mask:
        out_ref, mask_ref, m_ref, l_ref, acc_ref = refs
    else:
        out_ref, m_ref, l_ref, acc_ref = refs
    j = pl.program_id(1)

    @pl.when(j == 0)
    def _init():
        m_ref[...] = jnp.full_like(m_ref, -jnp.inf)
        l_ref[...] = jnp.zeros_like(l_ref)
        acc_ref[...] = jnp.zeros_like(acc_ref)

    h_blk = h_ref[...]
    row = jax.lax.broadcasted_iota(jnp.int32, h_blk.shape, 0) + j * bc
    h_blk = jnp.where(row < n, h_blk, 0.0)
    fd = jax.lax.dot_general(
        a_dst_ref[...], h_blk,
        dimension_numbers=(((1,), (1,)), ((), ())),
        preferred_element_type=jnp.float32,
    )
    s = f_src_ref[...] + fd
    e = jnp.where(s > 0, s, 0.2 * s)
    valid = adj_ref[...].astype(jnp.int32) > 0
    e = jnp.where(valid, e, jnp.float32(-1e9))
    if emit_mask:
        mask_ref[...] = valid.astype(jnp.int8)
    col = jax.lax.broadcasted_iota(jnp.int32, e.shape, 1) + j * bc
    e = jnp.where(col < n, e, -jnp.inf)

    m_prev = m_ref[...]
    m_new = jnp.maximum(m_prev, jnp.max(e, axis=1, keepdims=True))
    alpha = jnp.exp(m_prev - m_new)
    p = jnp.exp(e - m_new)
    l_ref[...] = l_ref[...] * alpha + jnp.sum(p, axis=1, keepdims=True)
    acc_ref[...] = acc_ref[...] * alpha + jnp.dot(
        p, h_blk, preferred_element_type=jnp.float32)
    m_ref[...] = m_new

    @pl.when(j == n_col_blocks - 1)
    def _finish():
        z = acc_ref[...] / l_ref[...]
        out_ref[...] = jnp.where(z > 0, z, jnp.exp(z) - 1.0)


def _flash_layer(f_src, h, a_dst, adj, emit_mask, block_rows, block_cols):
    N, D = h.shape
    nr = pl.cdiv(N, block_rows)
    nc = pl.cdiv(N, block_cols)
    out_shape = [jax.ShapeDtypeStruct((N, D), jnp.float32)]
    out_specs = [pl.BlockSpec((block_rows, D), lambda i, j: (i, 0))]
    if emit_mask:
        out_shape.append(jax.ShapeDtypeStruct((N, N), jnp.int8))
        out_specs.append(pl.BlockSpec((block_rows, block_cols),
                                      lambda i, j: (i, j)))
    body = functools.partial(_flash_body, N, block_cols, nc, emit_mask)
    res = pl.pallas_call(
        body,
        grid=(nr, nc),
        in_specs=[
            pl.BlockSpec((block_rows, 1), lambda i, j: (i, 0)),
            pl.BlockSpec((block_cols, D), lambda i, j: (j, 0)),
            pl.BlockSpec((1, D), lambda i, j: (0, 0)),
            pl.BlockSpec((block_rows, block_cols), lambda i, j: (i, j)),
        ],
        out_specs=out_specs,
        out_shape=out_shape,
        scratch_shapes=[
            pltpu.VMEM((block_rows, 1), jnp.float32),
            pltpu.VMEM((block_rows, 1), jnp.float32),
            pltpu.VMEM((block_rows, D), jnp.float32),
        ],
    )(f_src, h, a_dst.reshape(1, D), adj)
    return res if emit_mask else res[0]


def kernel(x, adj, W_e0, a_src_e0, a_dst_e0, W_d0, a_src_d0, a_dst_d0):
    h1, f1 = _proj(x, W_e0, a_src_e0, block_rows=512)
    h_enc, mask8 = _flash_layer(f1, h1, a_dst_e0, adj, emit_mask=True,
                                block_rows=256, block_cols=1024)
    h2, f2 = _proj(h_enc, W_d0, a_src_d0, block_rows=512)
    x_hat = _flash_layer(f2, h2, a_dst_d0, mask8, emit_mask=False,
                         block_rows=256, block_cols=1024)
    return (h_enc, x_hat)
